# Initial kernel scaffold; baseline (speedup 1.0000x reference)
import jax
import jax.numpy as jnp
from jax.experimental import pallas as pl


def kernel(input1, input2, embedding_table):
    raise NotImplementedError("write your pallas kernel here")


# speedup vs baseline: 179.2103x; 179.2103x over previous
import functools

import jax
import jax.numpy as jnp
from jax.experimental import pallas as pl


def _edit_kernel(a_ref, b_ref, t_ref, o_ref, *, L):
    gblk = a_ref.shape[1]
    shape = (gblk, 128)
    one = jnp.int32(1)
    a = [a_ref[j] for j in range(L)]

    Pv = jnp.full(shape, (1 << L) - 1, jnp.int32)
    Mv = jnp.zeros(shape, jnp.int32)
    score = jnp.full(shape, L, jnp.int32)
    for i in range(L):
        bi = b_ref[i]
        Eq = jnp.zeros(shape, jnp.int32)
        for j in range(L):
            Eq = Eq | jnp.where(a[j] == bi, jnp.int32(1 << j), jnp.int32(0))
        Xv = Eq | Mv
        Xh = (((Eq & Pv) + Pv) ^ Pv) | Eq
        Ph = Mv | ~(Xh | Pv)
        Mh = Pv & Xh
        score = score + ((Ph >> (L - 1)) & one) - ((Mh >> (L - 1)) & one)
        Ph = (Ph << 1) | one
        Mh = Mh << 1
        Pv = Mh | ~(Xv | Ph)
        Mv = Ph & Xv

    for d in range(4):
        acc = jnp.zeros(shape, jnp.float32)
        for k in range(L + 1):
            acc = jnp.where(score == k, t_ref[k, d], acc)
        o_ref[d] = acc


def kernel(input1, input2, embedding_table):
    B, L = input1.shape
    G = B // 128
    grid = 8
    gblk = G // grid
    a3 = input1.T.reshape(L, G, 128)
    b3 = input2.T.reshape(L, G, 128)
    out = pl.pallas_call(
        functools.partial(_edit_kernel, L=L),
        grid=(grid,),
        in_specs=[
            pl.BlockSpec((L, gblk, 128), lambda g: (0, g, 0)),
            pl.BlockSpec((L, gblk, 128), lambda g: (0, g, 0)),
            pl.BlockSpec((32, 4), lambda g: (0, 0)),
        ],
        out_specs=pl.BlockSpec((4, gblk, 128), lambda g: (0, g, 0)),
        out_shape=jax.ShapeDtypeStruct((4, G, 128), jnp.float32),
    )(a3, b3, embedding_table)
    return out.transpose(1, 2, 0).reshape(B, 4)
